# baseline (speedup 1.0000x reference)
import functools

import jax
import jax.numpy as jnp
import numpy as np
from jax import lax
from jax.experimental import pallas as pl
from jax.experimental.pallas import tpu as pltpu
from jax.experimental.pallas import tpu_sc as plsc

CUTOFF = 10.0
SHIFT = float(np.log(2.0))

NC = 2
NS = 16
NW = NC * NS

K = 80


def _ssp(x):
    return jax.nn.softplus(x) - SHIFT


def _filters_body(ea_ref, el_ref, w1_ref, b1_ref, w2_ref, b2_ref, out_ref):
    ea = ea_ref[...]
    t = jnp.dot(ea, w1_ref[0], preferred_element_type=jnp.float32) + b1_ref[0]
    t = _ssp(t)
    w = jnp.dot(t, w2_ref[0], preferred_element_type=jnp.float32) + b2_ref[0]
    el = el_ref[...]
    c = 0.5 * (jnp.cos(el * (np.pi / CUTOFF)) + 1.0)
    c = c * (el <= CUTOFF).astype(jnp.float32) * (el >= 0.0).astype(jnp.float32)
    w = w * c
    d = w.shape[1]
    wb = w.astype(jnp.bfloat16)
    lo = jax.lax.bitcast_convert_type(wb[:, :d // 2], jnp.uint16)
    hi = jax.lax.bitcast_convert_type(wb[:, d // 2:], jnp.uint16)
    word = lo.astype(jnp.uint32) | (hi.astype(jnp.uint32) << 16)
    word = jax.lax.bitcast_convert_type(word, jnp.int32)
    w3 = word.reshape(word.shape[0] // 2, 2, d // 2)
    out_ref[...] = jnp.concatenate([w3[:, 0, :], w3[:, 1, :]], axis=-1)


def _layer_filters(edge_attr, el2, w1, b1, w2, b2):
    D_EDGE, D = w1.shape
    E = edge_attr.shape[0]
    BE = 2000
    return pl.pallas_call(
        _filters_body,
        grid=(E // BE,),
        in_specs=[
            pl.BlockSpec((BE, D_EDGE), lambda e: (e, 0)),
            pl.BlockSpec((BE, 1), lambda e: (e, 0)),
            pl.BlockSpec((1, D_EDGE, D), lambda e: (0, 0, 0)),
            pl.BlockSpec((1, 1, D), lambda e: (0, 0, 0)),
            pl.BlockSpec((1, D, D), lambda e: (0, 0, 0)),
            pl.BlockSpec((1, 1, D), lambda e: (0, 0, 0)),
        ],
        out_specs=pl.BlockSpec((BE // 2, D), lambda e: (e, 0)),
        out_shape=jax.ShapeDtypeStruct((E // 2, D), jnp.int32),
    )(edge_attr, el2, w1.reshape(1, D_EDGE, D), b1.reshape(1, 1, D),
      w2.reshape(1, D, D), b2.reshape(1, 1, D))


def _matmul_body(x_ref, w_ref, o_ref):
    o_ref[...] = jnp.dot(x_ref[...], w_ref[...], preferred_element_type=jnp.float32)


def _tc_matmul(x, w):
    n, d = x.shape
    BN = 2000
    return pl.pallas_call(
        _matmul_body,
        grid=(n // BN,),
        in_specs=[
            pl.BlockSpec((BN, d), lambda i: (i, 0)),
            pl.BlockSpec((d, d), lambda i: (0, 0)),
        ],
        out_specs=pl.BlockSpec((BN, d), lambda i: (i, 0)),
        out_shape=jax.ShapeDtypeStruct((n, d), jnp.float32),
    )(x, w)


def _update_body(aggp_ref, h_ref, l2w_ref, l2b_ref, lw_ref, lb_ref, l1n_ref,
                 hn_ref, x1_ref):
    agg = aggp_ref[0] + aggp_ref[1]
    t = jnp.dot(agg, l2w_ref[...], preferred_element_type=jnp.float32) + l2b_ref[...]
    t = _ssp(t)
    out = jnp.dot(t, lw_ref[...], preferred_element_type=jnp.float32) + lb_ref[...]
    hn = h_ref[...] + out
    hn_ref[...] = hn
    x1_ref[...] = jnp.dot(hn, l1n_ref[...], preferred_element_type=jnp.float32)


def _tc_update(aggp, h, l2w, l2b, lw, lb, l1n):
    n, d = h.shape
    BN = 2000
    return pl.pallas_call(
        _update_body,
        grid=(n // BN,),
        in_specs=[
            pl.BlockSpec((2, BN, d), lambda i: (0, i, 0)),
            pl.BlockSpec((BN, d), lambda i: (i, 0)),
            pl.BlockSpec((d, d), lambda i: (0, 0)),
            pl.BlockSpec((1, d), lambda i: (0, 0)),
            pl.BlockSpec((d, d), lambda i: (0, 0)),
            pl.BlockSpec((1, d), lambda i: (0, 0)),
            pl.BlockSpec((d, d), lambda i: (0, 0)),
        ],
        out_specs=[
            pl.BlockSpec((BN, d), lambda i: (i, 0)),
            pl.BlockSpec((BN, d), lambda i: (i, 0)),
        ],
        out_shape=[
            jax.ShapeDtypeStruct((n, d), jnp.float32),
            jax.ShapeDtypeStruct((n, d), jnp.float32),
        ],
    )(aggp, h, l2w, l2b, lw, lb, l1n)


def _sc_message(x1, wp, sd):
    n, d = x1.shape
    e = sd.shape[0]
    k = K
    epw = e // NW
    chunks = epw // k
    nvec = d // 16
    ZB = 48
    NCOPY = 13
    rpw = ZB * NCOPY
    rem = n - rpw * NS

    mesh = plsc.VectorSubcoreMesh(core_axis_name="c", subcore_axis_name="s")

    @functools.partial(
        pl.kernel,
        out_type=jax.ShapeDtypeStruct((NC, n, d), jnp.float32),
        mesh=mesh,
        compiler_params=pltpu.CompilerParams(needs_layout_passes=False),
        scratch_types=[
            pltpu.VMEM((2, k), jnp.int32),
            pltpu.VMEM((2, k), jnp.int32),
            pltpu.VMEM((2, k), jnp.int32),
            pltpu.VMEM((2, k // 2, d), jnp.int32),
            pltpu.VMEM((2, k, d), jnp.float32),
            pltpu.VMEM((ZB, d), jnp.float32),
            pltpu.VMEM_SHARED((n, d), jnp.float32),
            pltpu.SemaphoreType.DMA,
            pltpu.SemaphoreType.DMA,
            pltpu.SemaphoreType.DMA,
        ],
    )
    def launch(x1_hbm, wp_hbm, sd_hbm, out_hbm,
               sd_v, idxs_v, idxd_v, w_v, x_v, z_v, acc_sh,
               isem, gsem, wsem):
        c = lax.axis_index("c")
        s = lax.axis_index("s")
        wid = s * NC + c
        base0 = wid * epw
        wrow0 = base0 // 2
        row0 = s * rpw

        def _zero(i, _):
            for j in range(nvec):
                z_v[i, pl.ds(j * 16, 16)] = jnp.zeros((16,), jnp.float32)
            return None
        lax.fori_loop(0, ZB, _zero, None)

        for kk in range(NCOPY):
            pltpu.sync_copy(z_v, acc_sh.at[pl.ds(row0 + kk * ZB, ZB)])

        @pl.when(s == NS - 1)
        def _():
            pltpu.sync_copy(z_v.at[pl.ds(0, rem)],
                            acc_sh.at[pl.ds(rpw * NS, rem)])

        plsc.subcore_barrier()

        def _start_sd(g):
            pltpu.async_copy(sd_hbm.at[pl.ds(base0 + g * k, k)],
                             sd_v.at[lax.rem(g, 2)], isem)

        def _wait_sd():
            pltpu.make_async_copy(sd_hbm.at[pl.ds(0, k)], sd_v.at[0],
                                  isem).wait()

        def _unpack_idx(g):
            b = lax.rem(g, 2)
            for v in range(k // 16):
                sl = pl.ds(v * 16, 16)
                p = sd_v[b, sl]
                idxs_v[b, sl] = lax.shift_right_logical(p, 16)
                idxd_v[b, sl] = lax.bitwise_and(p, jnp.int32(0xFFFF))

        def _start_fetch(g):
            b = lax.rem(g, 2)
            pltpu.async_copy(x1_hbm.at[idxs_v.at[b]], x_v.at[b], gsem)
            woff = pl.multiple_of(wrow0 + g * (k // 2), 8)
            pltpu.async_copy(wp_hbm.at[pl.ds(woff, k // 2)],
                             w_v.at[b], wsem)

        def _wait_fetch():
            pltpu.make_async_copy(x1_hbm.at[idxs_v.at[0]], x_v.at[0],
                                  gsem).wait()
            pltpu.make_async_copy(wp_hbm.at[pl.ds(0, k // 2)], w_v.at[0],
                                  wsem).wait()

        _start_sd(0)
        _wait_sd()
        _start_sd(1)
        _unpack_idx(0)
        _start_fetch(0)

        mask_hi = jnp.int32(-65536)

        def _edge_chunk(g, _):
            gb = lax.rem(g, 2)

            _wait_fetch()

            @pl.when(g + 1 < chunks)
            def _():
                _wait_sd()

                @pl.when(g + 2 < chunks)
                def _():
                    _start_sd(g + 2)
                _unpack_idx(g + 1)
                _start_fetch(g + 1)

            @plsc.parallel_loop(0, k // 2, unroll=2)
            def _(i2):
                for h in range(2):
                    row = 2 * i2 + h
                    for u in range(d // 32):
                        w32 = w_v[gb, i2, pl.ds(h * (d // 2) + u * 16, 16)]
                        lo = plsc.bitcast(lax.shift_left(w32, 16),
                                          jnp.float32)
                        hi = plsc.bitcast(lax.bitwise_and(w32, mask_hi),
                                          jnp.float32)
                        sl_lo = pl.ds(u * 16, 16)
                        sl_hi = pl.ds(d // 2 + u * 16, 16)
                        x_v[gb, row, sl_lo] = x_v[gb, row, sl_lo] * lo
                        x_v[gb, row, sl_hi] = x_v[gb, row, sl_hi] * hi

            pltpu.sync_copy(x_v.at[gb], acc_sh.at[idxd_v.at[gb]],
                            add=True)
            return None
        lax.fori_loop(0, chunks, _edge_chunk, None)

        plsc.subcore_barrier()
        for kk in range(NCOPY):
            pltpu.sync_copy(acc_sh.at[pl.ds(row0 + kk * ZB, ZB)],
                            out_hbm.at[c, pl.ds(row0 + kk * ZB, ZB)])

        @pl.when(s == NS - 1)
        def _():
            pltpu.sync_copy(acc_sh.at[pl.ds(rpw * NS, rem)],
                            out_hbm.at[c, pl.ds(rpw * NS, rem)])

    return launch(x1, wp, sd)


def kernel(z, edge_index, edge_length, edge_attr, mlp_w1, mlp_b1, mlp_w2,
           mlp_b2, lin1_w, lin2_w, lin2_b, lin_w, lin_b):
    L = mlp_w1.shape[0]
    E = edge_index.shape[1]
    sd = (edge_index[0] << 16) | edge_index[1]
    el2 = edge_length.reshape(E, 1)

    h = z
    x1 = _tc_matmul(z, lin1_w[0])
    for i in range(L):
        w_i = _layer_filters(edge_attr, el2, mlp_w1[i], mlp_b1[i],
                             mlp_w2[i], mlp_b2[i])
        aggp = _sc_message(x1, w_i, sd)
        l1n = lin1_w[(i + 1) % L]
        h, x1 = _tc_update(aggp, h, lin2_w[i], lin2_b[i].reshape(1, -1),
                           lin_w[i], lin_b[i].reshape(1, -1), l1n)
    return h


# speedup vs baseline: 1.0977x; 1.0977x over previous
import functools

import jax
import jax.numpy as jnp
import numpy as np
from jax import lax
from jax.experimental import pallas as pl
from jax.experimental.pallas import tpu as pltpu
from jax.experimental.pallas import tpu_sc as plsc

CUTOFF = 10.0
SHIFT = float(np.log(2.0))

NC = 2
NS = 16
NW = NC * NS

K = 80


def _ssp(x):
    return jax.nn.softplus(x) - SHIFT


def _round_bf16_bits(w):
    u = jax.lax.bitcast_convert_type(w, jnp.uint32)
    return jax.lax.shift_right_logical(u + jnp.uint32(0x8000),
                                       jnp.uint32(16))


def _pack_words(w):
    d = w.shape[1]
    lo = _round_bf16_bits(w[:, :d // 2])
    hi = _round_bf16_bits(w[:, d // 2:])
    word = lo | jax.lax.shift_left(hi, jnp.uint32(16))
    return jax.lax.bitcast_convert_type(word, jnp.int32)


def _filters_body(ea1_ref, ea2_ref, el1_ref, el2_ref, w1_ref, b1_ref,
                  w2_ref, b2_ref, out_ref):
    def filt(ea_ref, el_ref):
        ea = ea_ref[...]
        t = jnp.dot(ea, w1_ref[0], preferred_element_type=jnp.float32) + b1_ref[0]
        t = _ssp(t)
        w = jnp.dot(t, w2_ref[0], preferred_element_type=jnp.float32) + b2_ref[0]
        el = el_ref[...]
        c = 0.5 * (jnp.cos(el * (np.pi / CUTOFF)) + 1.0)
        c = c * (el <= CUTOFF).astype(jnp.float32) * (el >= 0.0).astype(jnp.float32)
        return w * c

    wlo = _pack_words(filt(ea1_ref, el1_ref))
    whi = _pack_words(filt(ea2_ref, el2_ref))
    out_ref[...] = jnp.concatenate([wlo, whi], axis=-1)


def _layer_filters(edge_attr, el2, w1, b1, w2, b2):
    D_EDGE, D = w1.shape
    E = edge_attr.shape[0]
    BE = 1000
    nb = E // 2 // BE
    return pl.pallas_call(
        _filters_body,
        grid=(nb,),
        in_specs=[
            pl.BlockSpec((BE, D_EDGE), lambda e: (e, 0)),
            pl.BlockSpec((BE, D_EDGE), lambda e: (e + nb, 0)),
            pl.BlockSpec((BE, 1), lambda e: (e, 0)),
            pl.BlockSpec((BE, 1), lambda e: (e + nb, 0)),
            pl.BlockSpec((1, D_EDGE, D), lambda e: (0, 0, 0)),
            pl.BlockSpec((1, 1, D), lambda e: (0, 0, 0)),
            pl.BlockSpec((1, D, D), lambda e: (0, 0, 0)),
            pl.BlockSpec((1, 1, D), lambda e: (0, 0, 0)),
        ],
        out_specs=pl.BlockSpec((BE, D), lambda e: (e, 0)),
        out_shape=jax.ShapeDtypeStruct((E // 2, D), jnp.int32),
    )(edge_attr, edge_attr, el2, el2, w1.reshape(1, D_EDGE, D),
      b1.reshape(1, 1, D), w2.reshape(1, D, D), b2.reshape(1, 1, D))


def _matmul_body(x_ref, w_ref, o_ref):
    o_ref[...] = jnp.dot(x_ref[...], w_ref[...], preferred_element_type=jnp.float32)


def _tc_matmul(x, w):
    n, d = x.shape
    BN = 2000
    return pl.pallas_call(
        _matmul_body,
        grid=(n // BN,),
        in_specs=[
            pl.BlockSpec((BN, d), lambda i: (i, 0)),
            pl.BlockSpec((d, d), lambda i: (0, 0)),
        ],
        out_specs=pl.BlockSpec((BN, d), lambda i: (i, 0)),
        out_shape=jax.ShapeDtypeStruct((n, d), jnp.float32),
    )(x, w)


def _update_body(aggp_ref, h_ref, l2w_ref, l2b_ref, lw_ref, lb_ref, l1n_ref,
                 hn_ref, x1_ref):
    agg = aggp_ref[0] + aggp_ref[1]
    t = jnp.dot(agg, l2w_ref[...], preferred_element_type=jnp.float32) + l2b_ref[...]
    t = _ssp(t)
    out = jnp.dot(t, lw_ref[...], preferred_element_type=jnp.float32) + lb_ref[...]
    hn = h_ref[...] + out
    hn_ref[...] = hn
    x1_ref[...] = jnp.dot(hn, l1n_ref[...], preferred_element_type=jnp.float32)


def _tc_update(aggp, h, l2w, l2b, lw, lb, l1n):
    n, d = h.shape
    BN = 2000
    return pl.pallas_call(
        _update_body,
        grid=(n // BN,),
        in_specs=[
            pl.BlockSpec((2, BN, d), lambda i: (0, i, 0)),
            pl.BlockSpec((BN, d), lambda i: (i, 0)),
            pl.BlockSpec((d, d), lambda i: (0, 0)),
            pl.BlockSpec((1, d), lambda i: (0, 0)),
            pl.BlockSpec((d, d), lambda i: (0, 0)),
            pl.BlockSpec((1, d), lambda i: (0, 0)),
            pl.BlockSpec((d, d), lambda i: (0, 0)),
        ],
        out_specs=[
            pl.BlockSpec((BN, d), lambda i: (i, 0)),
            pl.BlockSpec((BN, d), lambda i: (i, 0)),
        ],
        out_shape=[
            jax.ShapeDtypeStruct((n, d), jnp.float32),
            jax.ShapeDtypeStruct((n, d), jnp.float32),
        ],
    )(aggp, h, l2w, l2b, lw, lb, l1n)


def _sc_message(x1, wp, sd):
    n, d = x1.shape
    e = sd.shape[0]
    k = K
    epw = e // NW
    chunks = epw // k
    nvec = d // 16
    ZB = 48
    NCOPY = 13
    rpw = ZB * NCOPY
    rem = n - rpw * NS

    mesh = plsc.VectorSubcoreMesh(core_axis_name="c", subcore_axis_name="s")

    @functools.partial(
        pl.kernel,
        out_type=jax.ShapeDtypeStruct((NC, n, d), jnp.float32),
        mesh=mesh,
        compiler_params=pltpu.CompilerParams(needs_layout_passes=False),
        scratch_types=[
            pltpu.VMEM((2, k), jnp.int32),
            pltpu.VMEM((2, k), jnp.int32),
            pltpu.VMEM((2, k), jnp.int32),
            pltpu.VMEM((2, k // 2, d), jnp.int32),
            pltpu.VMEM((2, k, d), jnp.float32),
            pltpu.VMEM((ZB, d), jnp.float32),
            pltpu.VMEM_SHARED((n, d), jnp.float32),
            pltpu.SemaphoreType.DMA,
            pltpu.SemaphoreType.DMA,
            pltpu.SemaphoreType.DMA,
        ],
    )
    def launch(x1_hbm, wp_hbm, sd_hbm, out_hbm,
               sd_v, idxs_v, idxd_v, w_v, x_v, z_v, acc_sh,
               isem, gsem, wsem):
        c = lax.axis_index("c")
        s = lax.axis_index("s")
        wid = s * NC + c
        base0 = wid * epw
        wrow0 = base0 // 2
        row0 = s * rpw

        def _zero(i, _):
            for j in range(nvec):
                z_v[i, pl.ds(j * 16, 16)] = jnp.zeros((16,), jnp.float32)
            return None
        lax.fori_loop(0, ZB, _zero, None)

        for kk in range(NCOPY):
            pltpu.sync_copy(z_v, acc_sh.at[pl.ds(row0 + kk * ZB, ZB)])

        @pl.when(s == NS - 1)
        def _():
            pltpu.sync_copy(z_v.at[pl.ds(0, rem)],
                            acc_sh.at[pl.ds(rpw * NS, rem)])

        plsc.subcore_barrier()

        def _start_sd(g):
            pltpu.async_copy(sd_hbm.at[pl.ds(base0 + g * k, k)],
                             sd_v.at[lax.rem(g, 2)], isem)

        def _wait_sd():
            pltpu.make_async_copy(sd_hbm.at[pl.ds(0, k)], sd_v.at[0],
                                  isem).wait()

        def _unpack_idx(g):
            b = lax.rem(g, 2)
            for v in range(k // 16):
                sl = pl.ds(v * 16, 16)
                p = sd_v[b, sl]
                idxs_v[b, sl] = lax.shift_right_logical(p, 16)
                idxd_v[b, sl] = lax.bitwise_and(p, jnp.int32(0xFFFF))

        def _start_fetch(g):
            b = lax.rem(g, 2)
            pltpu.async_copy(x1_hbm.at[idxs_v.at[b]], x_v.at[b], gsem)
            woff = pl.multiple_of(wrow0 + g * (k // 2), 8)
            pltpu.async_copy(wp_hbm.at[pl.ds(woff, k // 2)],
                             w_v.at[b], wsem)

        def _wait_fetch():
            pltpu.make_async_copy(x1_hbm.at[idxs_v.at[0]], x_v.at[0],
                                  gsem).wait()
            pltpu.make_async_copy(wp_hbm.at[pl.ds(0, k // 2)], w_v.at[0],
                                  wsem).wait()

        _start_sd(0)
        _wait_sd()
        _start_sd(1)
        _unpack_idx(0)
        _start_fetch(0)

        mask_hi = jnp.int32(-65536)

        def _edge_chunk(g, _):
            gb = lax.rem(g, 2)

            _wait_fetch()

            @pl.when(g + 1 < chunks)
            def _():
                _wait_sd()

                @pl.when(g + 2 < chunks)
                def _():
                    _start_sd(g + 2)
                _unpack_idx(g + 1)
                _start_fetch(g + 1)

            @plsc.parallel_loop(0, k // 2, unroll=2)
            def _(i2):
                for h in range(2):
                    row = h * (k // 2) + i2
                    for u in range(d // 32):
                        w32 = w_v[gb, i2, pl.ds(h * (d // 2) + u * 16, 16)]
                        lo = plsc.bitcast(lax.shift_left(w32, 16),
                                          jnp.float32)
                        hi = plsc.bitcast(lax.bitwise_and(w32, mask_hi),
                                          jnp.float32)
                        sl_lo = pl.ds(u * 16, 16)
                        sl_hi = pl.ds(d // 2 + u * 16, 16)
                        x_v[gb, row, sl_lo] = x_v[gb, row, sl_lo] * lo
                        x_v[gb, row, sl_hi] = x_v[gb, row, sl_hi] * hi

            pltpu.sync_copy(x_v.at[gb], acc_sh.at[idxd_v.at[gb]],
                            add=True)
            return None
        lax.fori_loop(0, chunks, _edge_chunk, None)

        plsc.subcore_barrier()
        for kk in range(NCOPY):
            pltpu.sync_copy(acc_sh.at[pl.ds(row0 + kk * ZB, ZB)],
                            out_hbm.at[c, pl.ds(row0 + kk * ZB, ZB)])

        @pl.when(s == NS - 1)
        def _():
            pltpu.sync_copy(acc_sh.at[pl.ds(rpw * NS, rem)],
                            out_hbm.at[c, pl.ds(rpw * NS, rem)])

    return launch(x1, wp, sd)


def kernel(z, edge_index, edge_length, edge_attr, mlp_w1, mlp_b1, mlp_w2,
           mlp_b2, lin1_w, lin2_w, lin2_b, lin_w, lin_b):
    L = mlp_w1.shape[0]
    E = edge_index.shape[1]
    sd = (edge_index[0] << 16) | edge_index[1]
    half = K // 2
    sd = jnp.stack([sd[:E // 2].reshape(E // K, half),
                    sd[E // 2:].reshape(E // K, half)], axis=1).reshape(E)
    el2 = edge_length.reshape(E, 1)

    h = z
    x1 = _tc_matmul(z, lin1_w[0])
    for i in range(L):
        w_i = _layer_filters(edge_attr, el2, mlp_w1[i], mlp_b1[i],
                             mlp_w2[i], mlp_b2[i])
        aggp = _sc_message(x1, w_i, sd)
        l1n = lin1_w[(i + 1) % L]
        h, x1 = _tc_update(aggp, h, lin2_w[i], lin2_b[i].reshape(1, -1),
                           lin_w[i], lin_b[i].reshape(1, -1), l1n)
    return h


# speedup vs baseline: 2.7587x; 2.5131x over previous
import functools

import jax
import jax.numpy as jnp
import numpy as np
from jax import lax
from jax.experimental import pallas as pl
from jax.experimental.pallas import tpu as pltpu
from jax.experimental.pallas import tpu_sc as plsc

CUTOFF = 10.0
SHIFT = float(np.log(2.0))

NC = 2
NS = 16
NW = NC * NS

K = 80


def _ssp(x):
    return jnp.maximum(x, 0.0) + jnp.log(1.0 + jnp.exp(-jnp.abs(x))) - SHIFT


def _round_bf16_bits(w):
    u = jax.lax.bitcast_convert_type(w, jnp.uint32)
    return jax.lax.shift_right_logical(u + jnp.uint32(0x8000),
                                       jnp.uint32(16))


def _pack_words(w):
    d = w.shape[1]
    lo = _round_bf16_bits(w[:, :d // 2])
    hi = _round_bf16_bits(w[:, d // 2:])
    word = lo | jax.lax.shift_left(hi, jnp.uint32(16))
    return jax.lax.bitcast_convert_type(word, jnp.int32)


def _cutoff_body(el_ref, out_ref):
    el = el_ref[...]
    u = el * (np.pi / (2.0 * CUTOFF))
    t = u * u
    p = 1.0 + t * (-1.0 / 2 + t * (1.0 / 24 + t * (-1.0 / 720
                                                   + t * (1.0 / 40320))))
    cc = p * p
    cc = cc * (el <= CUTOFF).astype(jnp.float32)
    cc = cc * (el >= 0.0).astype(jnp.float32)
    out_ref[...] = cc


def _cutoff_envelope(edge_length):
    E = edge_length.shape[0]
    el = edge_length.reshape(E // 128, 128)
    return pl.pallas_call(
        _cutoff_body,
        out_shape=jax.ShapeDtypeStruct((E // 128, 128), jnp.float32),
    )(el).reshape(E, 1)


def _filters_body(ea1_ref, ea2_ref, c1_ref, c2_ref, w1_ref, b1_ref,
                  w2_ref, b2_ref, out_ref):
    def filt(ea_ref, c_ref):
        ea = ea_ref[...]
        t = jnp.dot(ea, w1_ref[0], preferred_element_type=jnp.float32) + b1_ref[0]
        t = _ssp(t)
        w = jnp.dot(t, w2_ref[0], preferred_element_type=jnp.float32) + b2_ref[0]
        return w * c_ref[...]

    wlo = _pack_words(filt(ea1_ref, c1_ref))
    whi = _pack_words(filt(ea2_ref, c2_ref))
    out_ref[...] = jnp.concatenate([wlo, whi], axis=-1)


def _layer_filters(edge_attr, cenv, w1, b1, w2, b2):
    D_EDGE, D = w1.shape
    E = edge_attr.shape[0]
    BE = 1000
    nb = E // 2 // BE
    return pl.pallas_call(
        _filters_body,
        grid=(nb,),
        in_specs=[
            pl.BlockSpec((BE, D_EDGE), lambda e: (e, 0)),
            pl.BlockSpec((BE, D_EDGE), lambda e: (e + nb, 0)),
            pl.BlockSpec((BE, 1), lambda e: (e, 0)),
            pl.BlockSpec((BE, 1), lambda e: (e + nb, 0)),
            pl.BlockSpec((1, D_EDGE, D), lambda e: (0, 0, 0)),
            pl.BlockSpec((1, 1, D), lambda e: (0, 0, 0)),
            pl.BlockSpec((1, D, D), lambda e: (0, 0, 0)),
            pl.BlockSpec((1, 1, D), lambda e: (0, 0, 0)),
        ],
        out_specs=pl.BlockSpec((BE, D), lambda e: (e, 0)),
        out_shape=jax.ShapeDtypeStruct((E // 2, D), jnp.int32),
    )(edge_attr, edge_attr, cenv, cenv, w1.reshape(1, D_EDGE, D),
      b1.reshape(1, 1, D), w2.reshape(1, D, D), b2.reshape(1, 1, D))


def _matmul_body(x_ref, w_ref, o_ref):
    o_ref[...] = jnp.dot(x_ref[...], w_ref[...], preferred_element_type=jnp.float32)


def _tc_matmul(x, w):
    n, d = x.shape
    BN = 2000
    return pl.pallas_call(
        _matmul_body,
        grid=(n // BN,),
        in_specs=[
            pl.BlockSpec((BN, d), lambda i: (i, 0)),
            pl.BlockSpec((d, d), lambda i: (0, 0)),
        ],
        out_specs=pl.BlockSpec((BN, d), lambda i: (i, 0)),
        out_shape=jax.ShapeDtypeStruct((n, d), jnp.float32),
    )(x, w)


def _update_body(aggp_ref, h_ref, l2w_ref, l2b_ref, lw_ref, lb_ref, l1n_ref,
                 hn_ref, x1_ref):
    agg = aggp_ref[0] + aggp_ref[1]
    t = jnp.dot(agg, l2w_ref[...], preferred_element_type=jnp.float32) + l2b_ref[...]
    t = _ssp(t)
    out = jnp.dot(t, lw_ref[...], preferred_element_type=jnp.float32) + lb_ref[...]
    hn = h_ref[...] + out
    hn_ref[...] = hn
    x1_ref[...] = jnp.dot(hn, l1n_ref[...], preferred_element_type=jnp.float32)


def _tc_update(aggp, h, l2w, l2b, lw, lb, l1n):
    n, d = h.shape
    BN = 2000
    return pl.pallas_call(
        _update_body,
        grid=(n // BN,),
        in_specs=[
            pl.BlockSpec((2, BN, d), lambda i: (0, i, 0)),
            pl.BlockSpec((BN, d), lambda i: (i, 0)),
            pl.BlockSpec((d, d), lambda i: (0, 0)),
            pl.BlockSpec((1, d), lambda i: (0, 0)),
            pl.BlockSpec((d, d), lambda i: (0, 0)),
            pl.BlockSpec((1, d), lambda i: (0, 0)),
            pl.BlockSpec((d, d), lambda i: (0, 0)),
        ],
        out_specs=[
            pl.BlockSpec((BN, d), lambda i: (i, 0)),
            pl.BlockSpec((BN, d), lambda i: (i, 0)),
        ],
        out_shape=[
            jax.ShapeDtypeStruct((n, d), jnp.float32),
            jax.ShapeDtypeStruct((n, d), jnp.float32),
        ],
    )(aggp, h, l2w, l2b, lw, lb, l1n)


def _sc_message(x1, wp, sd):
    n, d = x1.shape
    e = sd.shape[0]
    k = K
    epw = e // NW
    chunks = epw // k
    nvec = d // 16
    ZB = 48
    NCOPY = 13
    rpw = ZB * NCOPY
    rem = n - rpw * NS

    mesh = plsc.VectorSubcoreMesh(core_axis_name="c", subcore_axis_name="s")

    @functools.partial(
        pl.kernel,
        out_type=jax.ShapeDtypeStruct((NC, n, d), jnp.float32),
        mesh=mesh,
        compiler_params=pltpu.CompilerParams(needs_layout_passes=False),
        scratch_types=[
            pltpu.VMEM((2, k), jnp.int32),
            pltpu.VMEM((2, k), jnp.int32),
            pltpu.VMEM((2, k), jnp.int32),
            pltpu.VMEM((2, k // 2, d), jnp.int32),
            pltpu.VMEM((2, k, d), jnp.float32),
            pltpu.VMEM((ZB, d), jnp.float32),
            pltpu.VMEM_SHARED((n, d), jnp.float32),
            pltpu.SemaphoreType.DMA,
            pltpu.SemaphoreType.DMA,
            pltpu.SemaphoreType.DMA,
        ],
    )
    def launch(x1_hbm, wp_hbm, sd_hbm, out_hbm,
               sd_v, idxs_v, idxd_v, w_v, x_v, z_v, acc_sh,
               isem, gsem, wsem):
        c = lax.axis_index("c")
        s = lax.axis_index("s")
        wid = s * NC + c
        base0 = wid * epw
        wrow0 = base0 // 2
        row0 = s * rpw

        def _zero(i, _):
            for j in range(nvec):
                z_v[i, pl.ds(j * 16, 16)] = jnp.zeros((16,), jnp.float32)
            return None
        lax.fori_loop(0, ZB, _zero, None)

        for kk in range(NCOPY):
            pltpu.sync_copy(z_v, acc_sh.at[pl.ds(row0 + kk * ZB, ZB)])

        @pl.when(s == NS - 1)
        def _():
            pltpu.sync_copy(z_v.at[pl.ds(0, rem)],
                            acc_sh.at[pl.ds(rpw * NS, rem)])

        plsc.subcore_barrier()

        def _start_sd(g):
            pltpu.async_copy(sd_hbm.at[pl.ds(base0 + g * k, k)],
                             sd_v.at[lax.rem(g, 2)], isem)

        def _wait_sd():
            pltpu.make_async_copy(sd_hbm.at[pl.ds(0, k)], sd_v.at[0],
                                  isem).wait()

        def _unpack_idx(g):
            b = lax.rem(g, 2)
            for v in range(k // 16):
                sl = pl.ds(v * 16, 16)
                p = sd_v[b, sl]
                idxs_v[b, sl] = lax.shift_right_logical(p, 16)
                idxd_v[b, sl] = lax.bitwise_and(p, jnp.int32(0xFFFF))

        def _start_fetch(g):
            b = lax.rem(g, 2)
            pltpu.async_copy(x1_hbm.at[idxs_v.at[b]], x_v.at[b], gsem)
            woff = pl.multiple_of(wrow0 + g * (k // 2), 8)
            pltpu.async_copy(wp_hbm.at[pl.ds(woff, k // 2)],
                             w_v.at[b], wsem)

        def _wait_fetch():
            pltpu.make_async_copy(x1_hbm.at[idxs_v.at[0]], x_v.at[0],
                                  gsem).wait()
            pltpu.make_async_copy(wp_hbm.at[pl.ds(0, k // 2)], w_v.at[0],
                                  wsem).wait()

        _start_sd(0)
        _wait_sd()
        _start_sd(1)
        _unpack_idx(0)
        _start_fetch(0)

        mask_hi = jnp.int32(-65536)

        def _edge_chunk(g, _):
            gb = lax.rem(g, 2)

            _wait_fetch()

            @pl.when(g + 1 < chunks)
            def _():
                _wait_sd()

                @pl.when(g + 2 < chunks)
                def _():
                    _start_sd(g + 2)
                _unpack_idx(g + 1)
                _start_fetch(g + 1)

            @plsc.parallel_loop(0, k // 2, unroll=2)
            def _(i2):
                for h in range(2):
                    row = h * (k // 2) + i2
                    for u in range(d // 32):
                        w32 = w_v[gb, i2, pl.ds(h * (d // 2) + u * 16, 16)]
                        lo = plsc.bitcast(lax.shift_left(w32, 16),
                                          jnp.float32)
                        hi = plsc.bitcast(lax.bitwise_and(w32, mask_hi),
                                          jnp.float32)
                        sl_lo = pl.ds(u * 16, 16)
                        sl_hi = pl.ds(d // 2 + u * 16, 16)
                        x_v[gb, row, sl_lo] = x_v[gb, row, sl_lo] * lo
                        x_v[gb, row, sl_hi] = x_v[gb, row, sl_hi] * hi

            pltpu.sync_copy(x_v.at[gb], acc_sh.at[idxd_v.at[gb]],
                            add=True)
            return None
        lax.fori_loop(0, chunks, _edge_chunk, None)

        plsc.subcore_barrier()
        for kk in range(NCOPY):
            pltpu.sync_copy(acc_sh.at[pl.ds(row0 + kk * ZB, ZB)],
                            out_hbm.at[c, pl.ds(row0 + kk * ZB, ZB)])

        @pl.when(s == NS - 1)
        def _():
            pltpu.sync_copy(acc_sh.at[pl.ds(rpw * NS, rem)],
                            out_hbm.at[c, pl.ds(rpw * NS, rem)])

    return launch(x1, wp, sd)


def kernel(z, edge_index, edge_length, edge_attr, mlp_w1, mlp_b1, mlp_w2,
           mlp_b2, lin1_w, lin2_w, lin2_b, lin_w, lin_b):
    L = mlp_w1.shape[0]
    E = edge_index.shape[1]
    sd = (edge_index[0] << 16) | edge_index[1]
    half = K // 2
    sd = jnp.stack([sd[:E // 2].reshape(E // K, half),
                    sd[E // 2:].reshape(E // K, half)], axis=1).reshape(E)
    cenv = _cutoff_envelope(edge_length)

    h = z
    x1 = _tc_matmul(z, lin1_w[0])
    for i in range(L):
        w_i = _layer_filters(edge_attr, cenv, mlp_w1[i], mlp_b1[i],
                             mlp_w2[i], mlp_b2[i])
        aggp = _sc_message(x1, w_i, sd)
        l1n = lin1_w[(i + 1) % L]
        h, x1 = _tc_update(aggp, h, lin2_w[i], lin2_b[i].reshape(1, -1),
                           lin_w[i], lin_b[i].reshape(1, -1), l1n)
    return h
